# baseline (speedup 1.0000x reference)
import functools

import jax
import jax.numpy as jnp
from jax import lax
from jax.experimental import pallas as pl
from jax.experimental.pallas import tpu as pltpu
from jax.experimental.pallas import tpu_sc as plsc

_NC = 2
_NS = 16
_NW = _NC * _NS
_CH = 128

_HID = 64
_DEGW = 16


_NB = 4


def _sc_body(with_deg, n_acc, cpw, *refs):
    if with_deg:
        (g_hbm, src_hbm, dst_hbm, z64_hbm, z16_hbm, ones_hbm,
         agg_out, deg_out,
         src_all, dst_all, rows0, rows1, rows2, rows3,
         ones_v, acc_sh, deg_sh, sem0, sem1, sem2, sem3) = refs
    else:
        (g_hbm, src_hbm, dst_hbm, z64_hbm,
         agg_out,
         src_all, dst_all, rows0, rows1, rows2, rows3,
         acc_sh, sem0, sem1, sem2, sem3) = refs
    rows = (rows0, rows1, rows2, rows3)
    sems = (sem0, sem1, sem2, sem3)

    cid = lax.axis_index("c")
    sid = lax.axis_index("s")
    wid = cid * _NS + sid

    rows_per_tile = n_acc // _NS

    i0 = sid * rows_per_tile
    pltpu.sync_copy(z64_hbm.at[pl.ds(i0, rows_per_tile)],
                    acc_sh.at[pl.ds(i0, rows_per_tile)])
    pltpu.sync_copy(src_hbm.at[pl.ds(wid * cpw, cpw)], src_all)
    pltpu.sync_copy(dst_hbm.at[pl.ds(wid * cpw, cpw)], dst_all)
    if with_deg:
        pltpu.sync_copy(z16_hbm.at[pl.ds(i0, rows_per_tile)],
                        deg_sh.at[pl.ds(i0, rows_per_tile)])
        pltpu.sync_copy(ones_hbm, ones_v)
    plsc.subcore_barrier()

    def fire(j, b):
        pltpu.async_copy(g_hbm.at[src_all.at[j]], rows[b], sems[b])

    def drain(j, b):
        pltpu.make_async_copy(g_hbm.at[src_all.at[j]], rows[b], sems[b]).wait()
        pltpu.sync_copy(rows[b], acc_sh.at[dst_all.at[j]], add=True)
        if with_deg:
            pltpu.sync_copy(ones_v, deg_sh.at[dst_all.at[j]], add=True)

    for b in range(_NB):
        fire(b, b)

    def step(t, carry):
        for b in range(_NB):
            j = t * _NB + b
            drain(j, b)
            fire(j + _NB, b)
        return carry

    lax.fori_loop(0, cpw // _NB - 1, step, 0)
    for b in range(_NB):
        drain(cpw - _NB + b, b)
    plsc.subcore_barrier()

    pltpu.sync_copy(acc_sh.at[pl.ds(i0, rows_per_tile)],
                    agg_out.at[cid, pl.ds(i0, rows_per_tile)])
    if with_deg:
        pltpu.sync_copy(deg_sh.at[pl.ds(i0, rows_per_tile)],
                        deg_out.at[cid, pl.ds(i0, rows_per_tile)])


def _sc_segsum(g, src2d, dst2d, zeros64, zeros16, ones, with_deg):
    n_acc = zeros64.shape[0]
    cpw = src2d.shape[0] // _NW

    mesh = plsc.VectorSubcoreMesh(core_axis_name="c", subcore_axis_name="s")
    f32 = jnp.float32
    out_type = [jax.ShapeDtypeStruct((_NC, n_acc, _HID), f32)]
    scratch = [pltpu.VMEM((cpw, _CH), jnp.int32),
               pltpu.VMEM((cpw, _CH), jnp.int32)]
    scratch += [pltpu.VMEM((_CH, _HID), f32)] * _NB
    if with_deg:
        out_type.append(jax.ShapeDtypeStruct((_NC, n_acc, _DEGW), f32))
        scratch.append(pltpu.VMEM((_CH, _DEGW), f32))
    scratch.append(pltpu.VMEM_SHARED((n_acc, _HID), f32))
    if with_deg:
        scratch.append(pltpu.VMEM_SHARED((n_acc, _DEGW), f32))
    scratch += [pltpu.SemaphoreType.DMA] * _NB

    fn = pl.kernel(functools.partial(_sc_body, with_deg, n_acc, cpw),
                   out_type=tuple(out_type), mesh=mesh,
                   scratch_types=tuple(scratch),
                   compiler_params=pltpu.CompilerParams(
                       use_tc_tiling_on_sc=False))
    if with_deg:
        return fn(g, src2d, dst2d, zeros64, zeros16, ones)
    return fn(g, src2d, dst2d, zeros64)


def _tc_pre(x, z_q, Wn0, Ws0, b0):
    n, lat = x.shape
    bn = 1000

    def body(x_ref, zq_ref, wn_ref, ws_ref, b_ref, g_ref, s_ref):
        xb = x_ref[...]
        zq = zq_ref[...]
        g_ref[...] = (jnp.dot(xb, wn_ref[:lat], preferred_element_type=jnp.float32)
                      + jnp.dot(zq, wn_ref[lat:], preferred_element_type=jnp.float32))
        s_ref[...] = (jnp.dot(xb, ws_ref[:lat], preferred_element_type=jnp.float32)
                      + jnp.dot(zq, ws_ref[lat:], preferred_element_type=jnp.float32)
                      + b_ref[...])

    return pl.pallas_call(
        body,
        grid=(n // bn,),
        in_specs=[pl.BlockSpec((bn, lat), lambda i: (i, 0)),
                  pl.BlockSpec((1, lat), lambda i: (0, 0)),
                  pl.BlockSpec((2 * lat, _HID), lambda i: (0, 0)),
                  pl.BlockSpec((2 * lat, _HID), lambda i: (0, 0)),
                  pl.BlockSpec((1, _HID), lambda i: (0, 0))],
        out_specs=[pl.BlockSpec((bn, _HID), lambda i: (i, 0)),
                   pl.BlockSpec((bn, _HID), lambda i: (i, 0))],
        out_shape=[jax.ShapeDtypeStruct((n, _HID), jnp.float32)] * 2,
    )(x, z_q.reshape(1, lat), Wn0, Ws0, b0.reshape(1, _HID))


def _tc_mid(s_prev, aggp, degp, Wn, Ws, b):
    n = s_prev.shape[0]
    bn = 1000

    def body(s_ref, a_ref, d_ref, wn_ref, ws_ref, b_ref, g_ref, s_out_ref):
        agg = a_ref[0] + a_ref[1]
        deg = d_ref[0, :, :1] + d_ref[1, :, :1]
        rdeg = 1.0 / jnp.maximum(deg, 1.0)
        h = jnp.maximum(s_ref[...] + agg * rdeg, 0.0)
        g_ref[...] = jnp.dot(h, wn_ref[...], preferred_element_type=jnp.float32)
        s_out_ref[...] = (jnp.dot(h, ws_ref[...], preferred_element_type=jnp.float32)
                          + b_ref[...])

    return pl.pallas_call(
        body,
        grid=(n // bn,),
        in_specs=[pl.BlockSpec((bn, _HID), lambda i: (i, 0)),
                  pl.BlockSpec((_NC, bn, _HID), lambda i: (0, i, 0)),
                  pl.BlockSpec((_NC, bn, _DEGW), lambda i: (0, i, 0)),
                  pl.BlockSpec((_HID, _HID), lambda i: (0, 0)),
                  pl.BlockSpec((_HID, _HID), lambda i: (0, 0)),
                  pl.BlockSpec((1, _HID), lambda i: (0, 0))],
        out_specs=[pl.BlockSpec((bn, _HID), lambda i: (i, 0)),
                   pl.BlockSpec((bn, _HID), lambda i: (i, 0))],
        out_shape=[jax.ShapeDtypeStruct((n, _HID), jnp.float32)] * 2,
    )(s_prev, aggp, degp, Wn, Ws, b.reshape(1, _HID))


def _tc_final(s2, aggp, degp, pos, Sw0, Sb0, Sw1, Sb1):
    n = s2.shape[0]

    def body(s_ref, a_ref, d_ref, pos_ref, w0_ref, b0_ref, w1_ref, b1_ref,
             p_ref, w_ref):
        agg = a_ref[0] + a_ref[1]
        deg = d_ref[0, :, :1] + d_ref[1, :, :1]
        h = s_ref[...] + agg * (1.0 / jnp.maximum(deg, 1.0))
        m = jnp.maximum(jnp.dot(h, w0_ref[...], preferred_element_type=jnp.float32)
                        + b0_ref[...], 0.0)
        sc = jnp.dot(m, w1_ref[...], preferred_element_type=jnp.float32) + b1_ref[0, 0]
        e = jnp.exp(sc - jnp.max(sc))
        w = e / jnp.sum(e)
        w_ref[...] = w
        p_ref[...] = jnp.sum(w * pos_ref[...], axis=0, keepdims=True)

    return pl.pallas_call(
        body,
        out_shape=[jax.ShapeDtypeStruct((1, 2), jnp.float32),
                   jax.ShapeDtypeStruct((n, 1), jnp.float32)],
    )(s2, aggp, degp, pos, Sw0, Sb0.reshape(1, _HID), Sw1, Sb1.reshape(1, 1))


def kernel(x, pos, edge_index, z_q, Ws0, Wn0, b0, Ws1, Wn1, b1,
           Ws2, Wn2, b2, Sw0, Sb0, Sw1, Sb1):
    n = x.shape[0]
    e = edge_index.shape[1]
    n_acc = ((n + 1 + _NS * 8 - 1) // (_NS * 8)) * (_NS * 8)

    step = _NW * _CH * _NB
    e_pad = ((e + step - 1) // step) * step
    src = jnp.concatenate(
        [edge_index[0], jnp.zeros((e_pad - e,), jnp.int32)]).reshape(-1, _CH)
    dst = jnp.concatenate(
        [edge_index[1], jnp.full((e_pad - e,), n, jnp.int32)]).reshape(-1, _CH)

    zeros64 = jnp.zeros((n_acc, _HID), jnp.float32)
    zeros16 = jnp.zeros((n_acc, _DEGW), jnp.float32)
    ones = jnp.ones((_CH, _DEGW), jnp.float32)

    g0, s0 = _tc_pre(x, z_q, Wn0, Ws0, b0)
    agg0, degp = _sc_segsum(g0, src, dst, zeros64, zeros16, ones, True)
    agg0, degp = agg0[:, :n], degp[:, :n]
    g1, s1 = _tc_mid(s0, agg0, degp, Wn1, Ws1, b1)
    (agg1,) = _sc_segsum(g1, src, dst, zeros64, zeros16, ones, False)
    agg1 = agg1[:, :n]
    g2, s2 = _tc_mid(s1, agg1, degp, Wn2, Ws2, b2)
    (agg2,) = _sc_segsum(g2, src, dst, zeros64, zeros16, ones, False)
    agg2 = agg2[:, :n]
    p2d, w2d = _tc_final(s2, agg2, degp, pos, Sw0, Sb0, Sw1, Sb1)

    return (p2d.reshape(2), w2d.reshape(n))


# speedup vs baseline: 1.9126x; 1.9126x over previous
import functools

import jax
import jax.numpy as jnp
from jax import lax
from jax.experimental import pallas as pl
from jax.experimental.pallas import tpu as pltpu
from jax.experimental.pallas import tpu_sc as plsc

_NC = 2
_NS = 16
_NW = _NC * _NS
_CH = 128
_NB = 2

_LAT = 128
_HID = 64
_DEGW = 16


def _bdot(a, b):
    return jnp.dot(a.astype(jnp.bfloat16), b.astype(jnp.bfloat16),
                   preferred_element_type=jnp.float32)


def _sc_body(n_pad, cpw, *refs):
    h_hbm, src_hbm, dst_hbm, z64_hbm, agg_out, src_all, dst_all = refs[:7]
    rows = refs[7:7 + _NB]
    h_sh, acc_sh = refs[7 + _NB:9 + _NB]
    sems = refs[9 + _NB:9 + 2 * _NB]

    cid = lax.axis_index("c")
    sid = lax.axis_index("s")
    wid = cid * _NS + sid

    rpt = n_pad // _NS

    i0 = sid * rpt
    pltpu.sync_copy(h_hbm.at[pl.ds(i0, rpt)], h_sh.at[pl.ds(i0, rpt)])
    pltpu.sync_copy(z64_hbm.at[pl.ds(i0, rpt)], acc_sh.at[pl.ds(i0, rpt)])
    pltpu.sync_copy(src_hbm.at[pl.ds(wid * cpw, cpw)], src_all)
    pltpu.sync_copy(dst_hbm.at[pl.ds(wid * cpw, cpw)], dst_all)
    plsc.subcore_barrier()

    def fire(j, b):
        pltpu.async_copy(h_sh.at[src_all.at[j]], rows[b], sems[b])

    def drain(j, b):
        pltpu.make_async_copy(h_sh.at[src_all.at[j]], rows[b], sems[b]).wait()
        pltpu.sync_copy(rows[b], acc_sh.at[dst_all.at[j]], add=True)

    for b in range(_NB):
        fire(b, b)

    def step(t, carry):
        for b in range(_NB):
            j = t * _NB + b
            drain(j, b)
            fire(j + _NB, b)
        return carry

    lax.fori_loop(0, cpw // _NB - 1, step, 0)
    for b in range(_NB):
        drain(cpw - _NB + b, b)
    plsc.subcore_barrier()

    pltpu.sync_copy(acc_sh.at[pl.ds(i0, rpt)],
                    agg_out.at[cid, pl.ds(i0, rpt)])


def _sc_segsum(h, src2d, dst2d, zeros64):
    n_pad = h.shape[0]
    cpw = src2d.shape[0] // _NW

    mesh = plsc.VectorSubcoreMesh(core_axis_name="c", subcore_axis_name="s")
    f32 = jnp.float32
    out_type = [jax.ShapeDtypeStruct((_NC, n_pad, _HID), f32)]
    scratch = [pltpu.VMEM((cpw, _CH), jnp.int32),
               pltpu.VMEM((cpw, _CH), jnp.int32)]
    scratch += [pltpu.VMEM((_CH, _HID), f32)] * _NB
    scratch.append(pltpu.VMEM_SHARED((n_pad, _HID), f32))
    scratch.append(pltpu.VMEM_SHARED((n_pad, _HID), f32))
    scratch += [pltpu.SemaphoreType.DMA] * _NB

    fn = pl.kernel(functools.partial(_sc_body, n_pad, cpw),
                   out_type=tuple(out_type), mesh=mesh,
                   scratch_types=tuple(scratch),
                   compiler_params=pltpu.CompilerParams(
                       use_tc_tiling_on_sc=False))
    return fn(h, src2d, dst2d, zeros64)


def _sc_deg_body(n_pad, cpw, dst_hbm, z16_hbm, ones_hbm, deg_out,
                 dst_all, ones_v, deg_sh):
    cid = lax.axis_index("c")
    sid = lax.axis_index("s")
    wid = cid * _NS + sid
    rpt = n_pad // _NS
    i0 = sid * rpt

    pltpu.sync_copy(z16_hbm.at[pl.ds(i0, rpt)], deg_sh.at[pl.ds(i0, rpt)])
    pltpu.sync_copy(dst_hbm.at[pl.ds(wid * cpw, cpw)], dst_all)
    pltpu.sync_copy(ones_hbm, ones_v)
    plsc.subcore_barrier()

    def step(j, carry):
        pltpu.sync_copy(ones_v, deg_sh.at[dst_all.at[j]], add=True)
        return carry

    lax.fori_loop(0, cpw, step, 0)
    plsc.subcore_barrier()
    pltpu.sync_copy(deg_sh.at[pl.ds(i0, rpt)],
                    deg_out.at[cid, pl.ds(i0, rpt)])


def _sc_deg(dst2d, zeros16, ones, n_pad):
    cpw = dst2d.shape[0] // _NW
    mesh = plsc.VectorSubcoreMesh(core_axis_name="c", subcore_axis_name="s")
    f32 = jnp.float32
    fn = pl.kernel(functools.partial(_sc_deg_body, n_pad, cpw),
                   out_type=(jax.ShapeDtypeStruct((_NC, n_pad, _DEGW), f32),),
                   mesh=mesh,
                   scratch_types=(pltpu.VMEM((cpw, _CH), jnp.int32),
                                  pltpu.VMEM((_CH, _DEGW), f32),
                                  pltpu.VMEM_SHARED((n_pad, _DEGW), f32)),
                   compiler_params=pltpu.CompilerParams(
                       use_tc_tiling_on_sc=False))
    (deg,) = fn(dst2d, zeros16, ones)
    return deg


def _tc_l0(x, a0p, a1p, degp, z_q, Wn0, Ws0, b0):
    n_pad = x.shape[0]
    bn = n_pad // 16

    def body(x_ref, a0_ref, a1_ref, d_ref, zq_ref, wn_ref, ws_ref, b_ref,
             h_ref):
        deg = d_ref[0, :, :1] + d_ref[1, :, :1]
        cd = jnp.maximum(deg, 1.0)
        xb = x_ref[...]
        zq = zq_ref[...]
        m0 = (a0_ref[0] + a0_ref[1]) / cd
        m1 = (a1_ref[0] + a1_ref[1]) / cd
        self_t = (_bdot(xb, ws_ref[:_LAT]) + _bdot(zq, ws_ref[_LAT:]))
        zn = _bdot(zq, wn_ref[_LAT:])
        neigh = (_bdot(m0, wn_ref[:_HID]) + _bdot(m1, wn_ref[_HID:_LAT])
                 + jnp.where(deg > 0.0, zn, 0.0))
        h_ref[...] = jnp.maximum(self_t + neigh + b_ref[...], 0.0)

    return pl.pallas_call(
        body,
        grid=(n_pad // bn,),
        in_specs=[pl.BlockSpec((bn, _LAT), lambda i: (i, 0)),
                  pl.BlockSpec((_NC, bn, _HID), lambda i: (0, i, 0)),
                  pl.BlockSpec((_NC, bn, _HID), lambda i: (0, i, 0)),
                  pl.BlockSpec((_NC, bn, _DEGW), lambda i: (0, i, 0)),
                  pl.BlockSpec((1, _LAT), lambda i: (0, 0)),
                  pl.BlockSpec((2 * _LAT, _HID), lambda i: (0, 0)),
                  pl.BlockSpec((2 * _LAT, _HID), lambda i: (0, 0)),
                  pl.BlockSpec((1, _HID), lambda i: (0, 0))],
        out_specs=pl.BlockSpec((bn, _HID), lambda i: (i, 0)),
        out_shape=jax.ShapeDtypeStruct((n_pad, _HID), jnp.float32),
    )(x, a0p, a1p, degp, z_q.reshape(1, _LAT), Wn0, Ws0, b0.reshape(1, _HID))


def _tc_mid(h, aggp, degp, Wn, Ws, b):
    n_pad = h.shape[0]
    bn = n_pad // 16

    def body(h_ref, a_ref, d_ref, wn_ref, ws_ref, b_ref, o_ref):
        deg = d_ref[0, :, :1] + d_ref[1, :, :1]
        mean = (a_ref[0] + a_ref[1]) / jnp.maximum(deg, 1.0)
        o_ref[...] = jnp.maximum(
            _bdot(h_ref[...], ws_ref[...]) + _bdot(mean, wn_ref[...])
            + b_ref[...], 0.0)

    return pl.pallas_call(
        body,
        grid=(n_pad // bn,),
        in_specs=[pl.BlockSpec((bn, _HID), lambda i: (i, 0)),
                  pl.BlockSpec((_NC, bn, _HID), lambda i: (0, i, 0)),
                  pl.BlockSpec((_NC, bn, _DEGW), lambda i: (0, i, 0)),
                  pl.BlockSpec((_HID, _HID), lambda i: (0, 0)),
                  pl.BlockSpec((_HID, _HID), lambda i: (0, 0)),
                  pl.BlockSpec((1, _HID), lambda i: (0, 0))],
        out_specs=pl.BlockSpec((bn, _HID), lambda i: (i, 0)),
        out_shape=jax.ShapeDtypeStruct((n_pad, _HID), jnp.float32),
    )(h, aggp, degp, Wn, Ws, b.reshape(1, _HID))


def _tc_final(n, h, aggp, degp, pos, Wn2, Ws2, b2, Sw0, Sb0, Sw1, Sb1):
    n_pad = h.shape[0]

    def body(h_ref, a_ref, d_ref, pos_ref, wn_ref, ws_ref, b_ref,
             w0_ref, b0_ref, w1_ref, b1_ref, p_ref, w_ref):
        deg = d_ref[0, :, :1] + d_ref[1, :, :1]
        mean = (a_ref[0] + a_ref[1]) / jnp.maximum(deg, 1.0)
        h3 = (_bdot(h_ref[...], ws_ref[...]) + _bdot(mean, wn_ref[...])
              + b_ref[...])
        m = jnp.maximum(_bdot(h3, w0_ref[...]) + b0_ref[...], 0.0)
        sc = _bdot(m, w1_ref[...]) + b1_ref[0, 0]
        rid = lax.broadcasted_iota(jnp.int32, (n_pad, 1), 0)
        sc = jnp.where(rid < n, sc, -jnp.inf)
        e = jnp.exp(sc - jnp.max(sc))
        w = e / jnp.sum(e)
        w_ref[...] = w
        p_ref[...] = jnp.sum(w * pos_ref[...], axis=0, keepdims=True)

    return pl.pallas_call(
        body,
        out_shape=[jax.ShapeDtypeStruct((1, 2), jnp.float32),
                   jax.ShapeDtypeStruct((n_pad, 1), jnp.float32)],
    )(h, aggp, degp, pos, Wn2, Ws2, b2.reshape(1, _HID),
      Sw0, Sb0.reshape(1, _HID), Sw1, Sb1.reshape(1, 1))


def kernel(x, pos, edge_index, z_q, Ws0, Wn0, b0, Ws1, Wn1, b1,
           Ws2, Wn2, b2, Sw0, Sb0, Sw1, Sb1):
    n = x.shape[0]
    e = edge_index.shape[1]
    n_pad = ((n + 1 + _NS * 8 - 1) // (_NS * 8)) * (_NS * 8)

    step = _NW * _CH * _NB
    e_pad = ((e + step - 1) // step) * step
    src = jnp.concatenate(
        [edge_index[0], jnp.zeros((e_pad - e,), jnp.int32)]).reshape(-1, _CH)
    dst = jnp.concatenate(
        [edge_index[1], jnp.full((e_pad - e,), n, jnp.int32)]).reshape(-1, _CH)

    x_p = jnp.pad(x, ((0, n_pad - n), (0, 0)))
    pos_p = jnp.pad(pos, ((0, n_pad - n), (0, 0)))
    zeros64 = jnp.zeros((n_pad, _HID), jnp.float32)
    zeros16 = jnp.zeros((n_pad, _DEGW), jnp.float32)
    ones = jnp.ones((_CH, _DEGW), jnp.float32)

    degp = _sc_deg(dst, zeros16, ones, n_pad)

    (a0,) = _sc_segsum(x_p[:, :_HID], src, dst, zeros64)
    (a1,) = _sc_segsum(x_p[:, _HID:], src, dst, zeros64)
    h1 = _tc_l0(x_p, a0, a1, degp, z_q, Wn0, Ws0, b0)
    (agg1,) = _sc_segsum(h1, src, dst, zeros64)
    h2 = _tc_mid(h1, agg1, degp, Wn1, Ws1, b1)
    (agg2,) = _sc_segsum(h2, src, dst, zeros64)
    p2d, w2d = _tc_final(n, h2, agg2, degp, pos_p, Wn2, Ws2, b2,
                         Sw0, Sb0, Sw1, Sb1)

    return (p2d.reshape(2), w2d[:n].reshape(n))
